# baseline (speedup 1.0000x reference)
import functools

import jax
import jax.numpy as jnp
from jax import lax
from jax.experimental import pallas as pl
from jax.experimental.pallas import tpu as pltpu

B, S, H = 64, 2048, 768
E, HID, L = 64, 512, 4

BB = 8
SB = 512
G = 8


def _pool_body(h_ref, out_ref):
    j = pl.program_id(1)
    partial = jnp.sum(h_ref[...], axis=1)

    @pl.when(j == 0)
    def _():
        out_ref[...] = partial

    @pl.when(j > 0)
    def _():
        out_ref[...] = out_ref[...] + partial

    @pl.when(j == pl.num_programs(1) - 1)
    def _():
        pooled = out_ref[...] * (1.0 / S)
        mu = jnp.mean(pooled, axis=1, keepdims=True)
        var = jnp.mean((pooled - mu) ** 2, axis=1, keepdims=True)
        out_ref[...] = (pooled - mu) * jax.lax.rsqrt(var + 1e-5)


def _mlp_body(species_ref, xn_ref, g_ref, b_ref, w1_ref, b1_ref, w2_ref,
              b2_ref, out_ref):
    i = pl.program_id(0)

    acc = jnp.zeros((B, L), jnp.float32)
    xn = xn_ref[...]
    for g in range(G):
        e = i * G + g
        mask = species_ref[...] == e
        x = xn * g_ref[g, :, :] + b_ref[g, :, :]
        h = jnp.dot(x, w1_ref[g], preferred_element_type=jnp.float32)
        h = h + b1_ref[g, :, :]
        h = 0.5 * h * (1.0 + jax.lax.erf(h * 0.7071067811865476))
        logits = jax.lax.dot_general(
            h, w2_ref[g], (((1,), (1,)), ((), ())),
            preferred_element_type=jnp.float32)
        logits = logits + b2_ref[g, :, :]
        acc = acc + jnp.where(mask, logits, 0.0)

    @pl.when(i == 0)
    def _():
        out_ref[...] = acc

    @pl.when(i > 0)
    def _():
        out_ref[...] = out_ref[...] + acc


def kernel(hidden_states, species_idx, ln_g, ln_b, W1, b1, W2, b2):
    species_i32 = species_idx.astype(jnp.int32)

    xn = pl.pallas_call(
        _pool_body,
        grid=(B // BB, S // SB),
        in_specs=[pl.BlockSpec((BB, SB, H), lambda i, j: (i, j, 0))],
        out_specs=pl.BlockSpec((BB, H), lambda i, j: (i, 0)),
        out_shape=jax.ShapeDtypeStruct((B, H), jnp.float32),
    )(hidden_states)

    species2d = jnp.broadcast_to(species_i32.reshape(B, 1), (B, L))
    w2t = jnp.swapaxes(W2, 1, 2)

    logits = pl.pallas_call(
        _mlp_body,
        grid=(E // G,),
        in_specs=[
            pl.BlockSpec((B, L), lambda i: (0, 0)),
            pl.BlockSpec((B, H), lambda i: (0, 0)),
            pl.BlockSpec((G, 1, H), lambda i: (i, 0, 0)),
            pl.BlockSpec((G, 1, H), lambda i: (i, 0, 0)),
            pl.BlockSpec((G, H, HID), lambda i: (i, 0, 0)),
            pl.BlockSpec((G, 1, HID), lambda i: (i, 0, 0)),
            pl.BlockSpec((G, L, HID), lambda i: (i, 0, 0)),
            pl.BlockSpec((G, 1, L), lambda i: (i, 0, 0)),
        ],
        out_specs=pl.BlockSpec((B, L), lambda i: (0, 0)),
        out_shape=jax.ShapeDtypeStruct((B, L), jnp.float32),
    )(species2d, xn, ln_g.reshape(E, 1, H), ln_b.reshape(E, 1, H), W1,
      b1.reshape(E, 1, HID), w2t, b2.reshape(E, 1, L))
    return logits
